# baseline (speedup 1.0000x reference)
import jax
import jax.numpy as jnp
from jax.experimental import pallas as pl
from jax.experimental.pallas import tpu as pltpu

BATCH = 4096
NEIGHBOR_ITER = 4
NEIGHBOR_SIZE = 32
DIM = 64

ROWS = BATCH * NEIGHBOR_ITER
BLOCK_ROWS = 512


def _agg_kernel(nr_ref, nv_ref, ue_ref, sv_ref, w_ref, b_ref, out_ref):
    nr = nr_ref[...]
    nv = nv_ref[...]
    ue = ue_ref[...]
    sv = sv_ref[...]

    scores = jnp.sum(nr * ue[:, None, :], axis=-1) * (1.0 / DIM)
    m = jnp.max(scores, axis=-1, keepdims=True)
    e = jnp.exp(scores - m)
    w = e / jnp.sum(e, axis=-1, keepdims=True)

    agg = jnp.sum(w[..., None] * nv, axis=1) * (1.0 / NEIGHBOR_SIZE)
    x = sv + agg
    y = jnp.dot(x, w_ref[...], preferred_element_type=jnp.float32) + b_ref[...]
    out_ref[...] = jnp.maximum(y, 0.0)


def kernel(self_vectors, neighbor_vectors, neighbor_relations, user_embeddings, W, b, neighbor_size):
    nv = neighbor_vectors.reshape(ROWS, NEIGHBOR_SIZE, DIM)
    nr = neighbor_relations.reshape(ROWS, NEIGHBOR_SIZE, DIM)
    sv = self_vectors.reshape(ROWS, DIM)
    ue = jnp.broadcast_to(user_embeddings[:, None, :], (BATCH, NEIGHBOR_ITER, DIM)).reshape(ROWS, DIM)
    b2 = b.reshape(1, DIM)

    grid = (ROWS // BLOCK_ROWS,)
    out = pl.pallas_call(
        _agg_kernel,
        grid=grid,
        in_specs=[
            pl.BlockSpec((BLOCK_ROWS, NEIGHBOR_SIZE, DIM), lambda i: (i, 0, 0)),
            pl.BlockSpec((BLOCK_ROWS, NEIGHBOR_SIZE, DIM), lambda i: (i, 0, 0)),
            pl.BlockSpec((BLOCK_ROWS, DIM), lambda i: (i, 0)),
            pl.BlockSpec((BLOCK_ROWS, DIM), lambda i: (i, 0)),
            pl.BlockSpec((DIM, DIM), lambda i: (0, 0)),
            pl.BlockSpec((1, DIM), lambda i: (0, 0)),
        ],
        out_specs=pl.BlockSpec((BLOCK_ROWS, DIM), lambda i: (i, 0)),
        out_shape=jax.ShapeDtypeStruct((ROWS, DIM), jnp.float32),
        compiler_params=pltpu.CompilerParams(
            dimension_semantics=("arbitrary",),
        ),
    )(nr, nv, ue, sv, W, b2)
    return out.reshape(BATCH, NEIGHBOR_ITER, DIM)
